# baseline (speedup 1.0000x reference)
import functools

import jax
import jax.numpy as jnp
from jax import lax
from jax.experimental import pallas as pl
from jax.experimental.pallas import tpu as pltpu
from jax.experimental.pallas import tpu_sc as plsc

_C = 19
_EPS = 1e-06
_HW = 512 * 512
_NW = 32
_PXW = 4 * _HW // _NW
_CB = 2048
_NCH = 4

_mesh = plsc.VectorSubcoreMesh(core_axis_name="c", subcore_axis_name="s")


@functools.partial(
    pl.kernel,
    out_type=jax.ShapeDtypeStruct((_NW, 16), jnp.float32),
    mesh=_mesh,
    scratch_types=[
        pltpu.VMEM((2, _C, _CB), jnp.float32),
        pltpu.SemaphoreType.DMA,
        pltpu.SemaphoreType.DMA,
    ],
)
def _sc_probe(x_hbm, out_hbm, buf, sem0, sem1):
    w = lax.axis_index("s") * 2 + lax.axis_index("c")
    n = w // 8
    col0 = (w % 8) * _PXW
    sems = (sem0, sem1)

    def issue(k, b):
        return pltpu.async_copy(
            x_hbm.at[n, :, pl.ds(col0 + k * _CB, _CB)], buf.at[b], sems[b])

    hs = [None, None]
    hs[0] = issue(0, 0)
    for k in range(_NCH):
        b = k & 1
        if k + 1 < _NCH:
            hs[1 - b] = issue(k + 1, 1 - b)
        hs[b].wait()
    pltpu.sync_copy(buf.at[0, 0, pl.ds(0, 16)], out_hbm.at[w])


def _dice_body(tgt_ref, x_ref, out_ref, s_acc, i_acc, f_acc):
    n = pl.program_id(0)
    c = pl.program_id(1)
    num_n = pl.num_programs(0)
    num_c = pl.num_programs(1)

    v = x_ref[0, 0]
    t = tgt_ref[0]
    eqf = (t == c).astype(jnp.float32)
    psum = jnp.sum(v)
    inter = jnp.sum(v * eqf)
    freq = jnp.sum(eqf)

    @pl.when(n == 0)
    def _init():
        s_acc[c] = psum
        i_acc[c] = inter
        f_acc[c] = freq

    @pl.when(n != 0)
    def _accum():
        s_acc[c] = s_acc[c] + psum
        i_acc[c] = i_acc[c] + inter
        f_acc[c] = f_acc[c] + freq

    @pl.when((n == num_n - 1) & (c == num_c - 1))
    def _finish():
        def tot_body(k, acc):
            return acc + f_acc[k]
        tot_f = jax.lax.fori_loop(0, _C, tot_body, 0.0)

        def loss_body(k, acc):
            fk = f_acc[k]
            ik = i_acc[k]
            uk = s_acc[k] + fk - ik
            dice = 1.0 - (2.0 * ik + _EPS) / (uk + _EPS)
            w = tot_f / (fk * _C)
            return acc + dice * w
        out_ref[0, 0] = jax.lax.fori_loop(0, _C, loss_body, 0.0)


def kernel(inputs, targets):
    N, C, H, W = inputs.shape
    x3 = inputs.reshape(N, C, H * W)
    probe = _sc_probe(x3)
    out = pl.pallas_call(
        _dice_body,
        grid=(N, C),
        in_specs=[
            pl.BlockSpec((1, H, W), lambda n, c: (n, 0, 0)),
            pl.BlockSpec((1, 1, H, W), lambda n, c: (n, c, 0, 0)),
        ],
        out_specs=pl.BlockSpec(memory_space=pltpu.SMEM),
        out_shape=jax.ShapeDtypeStruct((1, 1), jnp.float32),
        scratch_shapes=[
            pltpu.SMEM((C,), jnp.float32),
            pltpu.SMEM((C,), jnp.float32),
            pltpu.SMEM((C,), jnp.float32),
        ],
    )(targets, inputs)
    return out[0, 0] + jnp.sum(probe) * 0.0


# speedup vs baseline: 5.7547x; 5.7547x over previous
import jax
import jax.numpy as jnp
from jax.experimental import pallas as pl
from jax.experimental.pallas import tpu as pltpu

_C = 19
_EPS = 1e-06


def _dice_body(tgt_ref, x_ref, out_ref, s_acc, i_acc, f_acc):
    n = pl.program_id(0)
    num_n = pl.num_programs(0)

    t = tgt_ref[0]
    for c in range(_C):
        v = x_ref[0, c]
        eqf = (t == c).astype(jnp.float32)
        psum = jnp.sum(v)
        inter = jnp.sum(v * eqf)
        freq = jnp.sum(eqf)

        @pl.when(n == 0)
        def _init(c=c, psum=psum, inter=inter, freq=freq):
            s_acc[c] = psum
            i_acc[c] = inter
            f_acc[c] = freq

        @pl.when(n != 0)
        def _accum(c=c, psum=psum, inter=inter, freq=freq):
            s_acc[c] = s_acc[c] + psum
            i_acc[c] = i_acc[c] + inter
            f_acc[c] = f_acc[c] + freq

    @pl.when(n == num_n - 1)
    def _finish():
        def tot_body(k, acc):
            return acc + f_acc[k]
        tot_f = jax.lax.fori_loop(0, _C, tot_body, 0.0)

        def loss_body(k, acc):
            fk = f_acc[k]
            ik = i_acc[k]
            uk = s_acc[k] + fk - ik
            dice = 1.0 - (2.0 * ik + _EPS) / (uk + _EPS)
            w = tot_f / (fk * _C)
            return acc + dice * w
        out_ref[0, 0] = jax.lax.fori_loop(0, _C, loss_body, 0.0)


def kernel(inputs, targets):
    N, C, H, W = inputs.shape
    out = pl.pallas_call(
        _dice_body,
        grid=(N,),
        in_specs=[
            pl.BlockSpec((1, H, W), lambda n: (n, 0, 0)),
            pl.BlockSpec((1, C, H, W), lambda n: (n, 0, 0, 0)),
        ],
        out_specs=pl.BlockSpec(memory_space=pltpu.SMEM),
        out_shape=jax.ShapeDtypeStruct((1, 1), jnp.float32),
        scratch_shapes=[
            pltpu.SMEM((_C,), jnp.float32),
            pltpu.SMEM((_C,), jnp.float32),
            pltpu.SMEM((_C,), jnp.float32),
        ],
    )(targets, inputs)
    return out[0, 0]


# speedup vs baseline: 6.0249x; 1.0470x over previous
import jax
import jax.numpy as jnp
from jax.experimental import pallas as pl
from jax.experimental.pallas import tpu as pltpu

_C = 19
_EPS = 1e-06


def _dice_body(tgt_ref, x_ref, out_ref, s_acc, i_acc, f_acc):
    n = pl.program_id(0)
    h = pl.program_id(1)
    num_n = pl.num_programs(0)
    num_h = pl.num_programs(1)

    t = tgt_ref[0]
    first = (n == 0) & (h == 0)
    for c in range(_C):
        v = x_ref[0, c]
        eq = t == c
        psum = jnp.sum(v)
        inter = jnp.sum(jnp.where(eq, v, 0.0))
        freq = jnp.sum(jnp.where(eq, 1.0, 0.0))

        @pl.when(first)
        def _init(c=c, psum=psum, inter=inter, freq=freq):
            s_acc[c] = psum
            i_acc[c] = inter
            f_acc[c] = freq

        @pl.when(jnp.logical_not(first))
        def _accum(c=c, psum=psum, inter=inter, freq=freq):
            s_acc[c] = s_acc[c] + psum
            i_acc[c] = i_acc[c] + inter
            f_acc[c] = f_acc[c] + freq

    @pl.when((n == num_n - 1) & (h == num_h - 1))
    def _finish():
        def tot_body(k, acc):
            return acc + f_acc[k]
        tot_f = jax.lax.fori_loop(0, _C, tot_body, 0.0)

        def loss_body(k, acc):
            fk = f_acc[k]
            ik = i_acc[k]
            uk = s_acc[k] + fk - ik
            dice = 1.0 - (2.0 * ik + _EPS) / (uk + _EPS)
            w = tot_f / (fk * _C)
            return acc + dice * w
        out_ref[0, 0] = jax.lax.fori_loop(0, _C, loss_body, 0.0)


def kernel(inputs, targets):
    N, C, H, W = inputs.shape
    HB = H // 2
    out = pl.pallas_call(
        _dice_body,
        grid=(N, 2),
        in_specs=[
            pl.BlockSpec((1, HB, W), lambda n, h: (n, h, 0)),
            pl.BlockSpec((1, C, HB, W), lambda n, h: (n, 0, h, 0)),
        ],
        out_specs=pl.BlockSpec(memory_space=pltpu.SMEM),
        out_shape=jax.ShapeDtypeStruct((1, 1), jnp.float32),
        scratch_shapes=[
            pltpu.SMEM((_C,), jnp.float32),
            pltpu.SMEM((_C,), jnp.float32),
            pltpu.SMEM((_C,), jnp.float32),
        ],
    )(targets, inputs)
    return out[0, 0]
